# Initial kernel scaffold; baseline (speedup 1.0000x reference)
import jax
import jax.numpy as jnp
from jax.experimental import pallas as pl


def kernel(x, W1, b1, W2, b2, k, training):
    raise NotImplementedError("write your pallas kernel here")


# speedup vs baseline: 1.9557x; 1.9557x over previous
import functools

import jax
import jax.numpy as jnp
from jax.experimental import pallas as pl
from jax.experimental.pallas import tpu as pltpu

N_TOKENS = 32768
N_EMBD = 4096
N_CHOICES = 64
HIDDEN = N_EMBD // 2

BT = 256


def _router_body(k_ref, x_ref, w1_ref, b1_ref, w2_ref, b2_ref, o_ref):
    h = jnp.dot(x_ref[...], w1_ref[...], preferred_element_type=jnp.float32)
    h = jnp.maximum(h + b1_ref[...], 0.0)
    logits = jnp.dot(h, w2_ref[...], preferred_element_type=jnp.float32)
    logits = logits + b2_ref[...]

    col = jax.lax.broadcasted_iota(jnp.int32, logits.shape, 1)
    big = jnp.int32(N_CHOICES)
    m1 = jnp.max(logits, axis=-1, keepdims=True)
    i1 = jnp.min(jnp.where(logits == m1, col, big), axis=-1, keepdims=True)
    neg = jnp.float32(-jnp.inf)
    masked = jnp.where(col == i1, neg, logits)
    m2 = jnp.max(masked, axis=-1, keepdims=True)
    i2 = jnp.min(jnp.where(masked == m2, col, big), axis=-1, keepdims=True)

    p1 = jax.nn.sigmoid(m1 - m2)
    k_is_1 = k_ref[0] == 1
    v1 = jnp.where(k_is_1, jnp.float32(1.0), p1)
    v2 = jnp.where(k_is_1, jnp.float32(0.0), 1.0 - p1)
    o_ref[...] = jnp.where(col == i1, v1, jnp.where(col == i2, v2, 0.0))


@functools.partial(jax.jit, static_argnames=("interpret",))
def _router(x, W1, b1, W2, b2, k, interpret=False):
    grid = (N_TOKENS // BT,)
    return pl.pallas_call(
        _router_body,
        grid=grid,
        in_specs=[
            pl.BlockSpec(memory_space=pltpu.SMEM),
            pl.BlockSpec((BT, N_EMBD), lambda i: (i, 0)),
            pl.BlockSpec((N_EMBD, HIDDEN), lambda i: (0, 0)),
            pl.BlockSpec((1, HIDDEN), lambda i: (0, 0)),
            pl.BlockSpec((HIDDEN, N_CHOICES), lambda i: (0, 0)),
            pl.BlockSpec((1, N_CHOICES), lambda i: (0, 0)),
        ],
        out_specs=pl.BlockSpec((BT, N_CHOICES), lambda i: (i, 0)),
        out_shape=jax.ShapeDtypeStruct((N_TOKENS, N_CHOICES), jnp.float32),
        interpret=interpret,
    )(k, x, W1, b1, W2, b2)


def kernel(x, W1, b1, W2, b2, k, training):
    k_arr = jnp.asarray(k, jnp.int32).reshape((1,))
    return _router(
        x, W1, b1.reshape(1, HIDDEN), W2, b2.reshape(1, N_CHOICES), k_arr
    )
